# baseline (speedup 1.0000x reference)
import math

import jax
import jax.numpy as jnp
from jax import lax
from jax.experimental import pallas as pl
from jax.experimental.pallas import tpu as pltpu
from jax.experimental.pallas import tpu_sc as plsc

EMBED = 128
NIDX = 23
SCALE = math.sqrt(EMBED)
NC, NS, L = 2, 16, 16
NW = NC * NS
N_TOKENS = 16384 * 100
TPW = N_TOKENS // NW
CH = 128
NCHUNK = TPW // CH
NBUF = 4
XB = 3200
NXB = TPW // XB


def _table_body(tok_ref, pos_ref, out_ref):
    tok = tok_ref[...]
    pos = pos_ref[...]
    out_ref[...] = (tok[:, None, :] + pos[None, :, :]) * SCALE


def _build_table(token_table, pos23):
    return pl.pallas_call(
        _table_body,
        out_shape=jax.ShapeDtypeStruct((NIDX, NIDX, EMBED), jnp.float32),
    )(token_table, pos23)


def _gather_body(x0_hbm, x1_hbm, tab_hbm, out_hbm,
                 xv0, xv1, idxa, r0, r1, r2, r3,
                 sg0, sg1, sg2, sg3, ss0, ss1, ss2, ss3):
    sid = lax.axis_index("s")
    wid = sid * NC + lax.axis_index("c")
    base_w = wid * TPW
    rows = [r0, r1, r2, r3]
    sgs = [sg0, sg1, sg2, sg3]
    sss = [ss0, ss1, ss2, ss3]

    def xblk(bi, _):
        blk = base_w + bi * XB
        pltpu.sync_copy(x0_hbm.at[pl.ds(blk, XB)], xv0)
        pltpu.sync_copy(x1_hbm.at[pl.ds(blk, XB)], xv1)

        def vrow(v, _):
            o = v * L
            idxa[pl.ds(bi * XB + o, L)] = (
                xv0[pl.ds(o, L)] * NIDX + xv1[pl.ds(o, L)]
            )
            return 0

        lax.fori_loop(0, XB // L, vrow, 0)
        return 0

    lax.fori_loop(0, NXB, xblk, 0)

    def idxs(i):
        return idxa.at[pl.ds(i * CH, CH)]

    def fire_g(i, b):
        pltpu.async_copy(tab_hbm.at[idxs(i)], rows[b], sgs[b])

    def wait_g(i, b):
        pltpu.make_async_copy(tab_hbm.at[idxs(i)], rows[b], sgs[b]).wait()

    def fire_s(i, b):
        pltpu.async_copy(rows[b], out_hbm.at[pl.ds(base_w + i * CH, CH)],
                         sss[b])

    def wait_s(i, b):
        pltpu.make_async_copy(rows[b],
                              out_hbm.at[pl.ds(base_w + i * CH, CH)],
                              sss[b]).wait()

    fire_g(0, 0)
    fire_g(1, 1)
    fire_g(2, 2)
    fire_g(3, 3)
    wait_g(0, 0)
    fire_s(0, 0)
    for i in range(4, 8):
        b = i % NBUF
        wait_s(i - 4, b)
        fire_g(i, b)
        wait_g(i - 3, (i - 3) % NBUF)
        fire_s(i - 3, (i - 3) % NBUF)

    def quad(it, _):
        base = 8 + it * NBUF
        for u in range(NBUF):
            i = base + u
            wait_s(i - 4, u)
            fire_g(i, u)
            wait_g(i - 3, (u + 1) % NBUF)
            fire_s(i - 3, (u + 1) % NBUF)
        return 0

    lax.fori_loop(0, (NCHUNK - 8) // NBUF, quad, 0)

    for i in range(NCHUNK - 3, NCHUNK):
        b = i % NBUF
        wait_g(i, b)
        fire_s(i, b)
    for i in range(NCHUNK - 4, NCHUNK):
        wait_s(i, i % NBUF)


def _gather(x0, x1, tab_flat):
    mesh = plsc.VectorSubcoreMesh(core_axis_name="c", subcore_axis_name="s")
    f = pl.kernel(
        _gather_body,
        out_type=jax.ShapeDtypeStruct((N_TOKENS, EMBED), jnp.float32),
        mesh=mesh,
        scratch_types=[
            pltpu.VMEM((XB,), jnp.int32),
            pltpu.VMEM((XB,), jnp.int32),
            pltpu.VMEM((NCHUNK * CH,), jnp.int32),
            pltpu.VMEM((CH, EMBED), jnp.float32),
            pltpu.VMEM((CH, EMBED), jnp.float32),
            pltpu.VMEM((CH, EMBED), jnp.float32),
            pltpu.VMEM((CH, EMBED), jnp.float32),
            pltpu.SemaphoreType.DMA,
            pltpu.SemaphoreType.DMA,
            pltpu.SemaphoreType.DMA,
            pltpu.SemaphoreType.DMA,
            pltpu.SemaphoreType.DMA,
            pltpu.SemaphoreType.DMA,
            pltpu.SemaphoreType.DMA,
            pltpu.SemaphoreType.DMA,
        ],
    )
    return f(x0, x1, tab_flat)


def kernel(x, token_table, pos_table):
    x2d = x.astype(jnp.int32).reshape(N_TOKENS, 2)
    x0 = x2d[:, 0]
    x1 = x2d[:, 1]
    tab = _build_table(token_table, pos_table[:NIDX])
    out = _gather(x0, x1, tab.reshape(NIDX * NIDX, EMBED))
    return out.reshape(16384, 100, EMBED)


# speedup vs baseline: 1.3923x; 1.3923x over previous
import math

import jax
import jax.numpy as jnp
from jax import lax
from jax.experimental import pallas as pl
from jax.experimental.pallas import tpu as pltpu
from jax.experimental.pallas import tpu_sc as plsc

EMBED = 128
NIDX = 23
SCALE = math.sqrt(EMBED)
NC, NS, L = 2, 16, 16
NW = NC * NS
N_TOKENS = 16384 * 100
TPW = N_TOKENS // NW
CH = 256
NCHUNK = TPW // CH
NBUF = 2
XB = 3200
NXB = TPW // XB


def _table_body(tok_ref, pos_ref, out_ref):
    tok = tok_ref[...]
    pos = pos_ref[...]
    out_ref[...] = (tok[:, None, :] + pos[None, :, :]) * SCALE


def _build_table(token_table, pos23):
    return pl.pallas_call(
        _table_body,
        out_shape=jax.ShapeDtypeStruct((NIDX, NIDX, EMBED), jnp.float32),
    )(token_table, pos23)


def _gather_body(x0_hbm, x1_hbm, tab_hbm, out_hbm,
                 xv0, xv1, idxa, tab_sp, *rest):
    rows = list(rest[:NBUF])
    sgs = list(rest[NBUF:2 * NBUF])
    sss = list(rest[2 * NBUF:3 * NBUF])
    sid = lax.axis_index("s")
    wid = sid * NC + lax.axis_index("c")
    base_w = wid * TPW

    @pl.when(sid == 0)
    def _():
        pltpu.sync_copy(tab_hbm, tab_sp)

    def xblk(bi, _):
        blk = base_w + bi * XB
        pltpu.sync_copy(x0_hbm.at[pl.ds(blk, XB)], xv0)
        pltpu.sync_copy(x1_hbm.at[pl.ds(blk, XB)], xv1)

        def vrow(v, _):
            o = v * L
            idxa[pl.ds(bi * XB + o, L)] = (
                xv0[pl.ds(o, L)] * NIDX + xv1[pl.ds(o, L)]
            )
            return 0

        lax.fori_loop(0, XB // L, vrow, 0)
        return 0

    lax.fori_loop(0, NXB, xblk, 0)

    plsc.subcore_barrier()

    def idxs(i):
        return idxa.at[pl.ds(i * CH, CH)]

    def fire_g(i, b):
        pltpu.async_copy(tab_sp.at[idxs(i)], rows[b], sgs[b])

    def wait_g(i, b):
        pltpu.make_async_copy(tab_sp.at[idxs(i)], rows[b], sgs[b]).wait()

    def fire_s(i, b):
        pltpu.async_copy(rows[b], out_hbm.at[pl.ds(base_w + i * CH, CH)],
                         sss[b])

    def wait_s(i, b):
        pltpu.make_async_copy(rows[b],
                              out_hbm.at[pl.ds(base_w + i * CH, CH)],
                              sss[b]).wait()

    for u in range(NBUF):
        fire_g(u, u)

    nround = NCHUNK // NBUF

    def roundfn(r, _):
        base = r * NBUF
        for u in range(NBUF):
            wait_g(base + u, u)
            fire_s(base + u, u)
        for u in range(NBUF):
            wait_s(base + u, u)
            fire_g(base + u + NBUF, u)
        return 0

    lax.fori_loop(0, nround - 1, roundfn, 0)

    base = (nround - 1) * NBUF
    for u in range(NBUF):
        wait_g(base + u, u)
        fire_s(base + u, u)
    for u in range(NBUF):
        wait_s(base + u, u)


def _gather(x0, x1, tab_flat):
    mesh = plsc.VectorSubcoreMesh(core_axis_name="c", subcore_axis_name="s")
    scratch = [
        pltpu.VMEM((XB,), jnp.int32),
        pltpu.VMEM((XB,), jnp.int32),
        pltpu.VMEM((NCHUNK * CH,), jnp.int32),
        pltpu.VMEM_SHARED((NIDX * NIDX, EMBED), jnp.float32),
    ]
    scratch += [pltpu.VMEM((CH, EMBED), jnp.float32)] * NBUF
    scratch += [pltpu.SemaphoreType.DMA] * (2 * NBUF)
    f = pl.kernel(
        _gather_body,
        out_type=jax.ShapeDtypeStruct((N_TOKENS, EMBED), jnp.float32),
        mesh=mesh,
        scratch_types=scratch,
    )
    return f(x0, x1, tab_flat)


def kernel(x, token_table, pos_table):
    x2d = x.astype(jnp.int32).reshape(N_TOKENS, 2)
    x0 = x2d[:, 0]
    x1 = x2d[:, 1]
    tab = _build_table(token_table, pos_table[:NIDX])
    out = _gather(x0, x1, tab.reshape(NIDX * NIDX, EMBED))
    return out.reshape(16384, 100, EMBED)


# speedup vs baseline: 1.5293x; 1.0984x over previous
import math

import jax
import jax.numpy as jnp
from jax import lax
from jax.experimental import pallas as pl
from jax.experimental.pallas import tpu as pltpu
from jax.experimental.pallas import tpu_sc as plsc

EMBED = 128
NIDX = 23
SCALE = math.sqrt(EMBED)
NC, NS, L = 2, 16, 16
NW = NC * NS
N_TOKENS = 16384 * 100
TPW = N_TOKENS // NW
CH = 128
NCHUNK = TPW // CH
NBUF = 4
XB = 3200
NXB = TPW // XB


def _table_body(tok_ref, pos_ref, out_ref):
    tok = tok_ref[...]
    pos = pos_ref[...]
    out_ref[...] = (tok[:, None, :] + pos[None, :, :]) * SCALE


def _build_table(token_table, pos23):
    return pl.pallas_call(
        _table_body,
        out_shape=jax.ShapeDtypeStruct((NIDX, NIDX, EMBED), jnp.float32),
    )(token_table, pos23)


def _gather_body(x0_hbm, x1_hbm, tab_hbm, out_hbm,
                 xv0, xv1, idxa, tab_sp, *rest):
    rows = list(rest[:NBUF])
    sgs = list(rest[NBUF:2 * NBUF])
    sss = list(rest[2 * NBUF:3 * NBUF])
    sid = lax.axis_index("s")
    wid = sid * NC + lax.axis_index("c")
    base_w = wid * TPW

    @pl.when(sid == 0)
    def _():
        pltpu.sync_copy(tab_hbm, tab_sp)

    def xblk(bi, _):
        blk = base_w + bi * XB
        pltpu.sync_copy(x0_hbm.at[pl.ds(blk, XB)], xv0)
        pltpu.sync_copy(x1_hbm.at[pl.ds(blk, XB)], xv1)

        def vrow(v, _):
            o = v * L
            idxa[pl.ds(bi * XB + o, L)] = (
                xv0[pl.ds(o, L)] * NIDX + xv1[pl.ds(o, L)]
            )
            return 0

        lax.fori_loop(0, XB // L, vrow, 0)
        return 0

    lax.fori_loop(0, NXB, xblk, 0)

    plsc.subcore_barrier()

    def idxs(i):
        return idxa.at[pl.ds(i * CH, CH)]

    def fire_g(i, b):
        pltpu.async_copy(tab_sp.at[idxs(i)], rows[b], sgs[b])

    def wait_g(i, b):
        pltpu.make_async_copy(tab_sp.at[idxs(i)], rows[b], sgs[b]).wait()

    def fire_s(i, b):
        pltpu.async_copy(rows[b], out_hbm.at[pl.ds(base_w + i * CH, CH)],
                         sss[b])

    def wait_s(i, b):
        pltpu.make_async_copy(rows[b],
                              out_hbm.at[pl.ds(base_w + i * CH, CH)],
                              sss[b]).wait()

    for u in range(NBUF):
        fire_g(u, u)

    nround = NCHUNK // NBUF

    def roundfn(r, _):
        base = r * NBUF
        for u in range(NBUF):
            wait_g(base + u, u)
            fire_s(base + u, u)
        for u in range(NBUF):
            wait_s(base + u, u)
            fire_g(base + u + NBUF, u)
        return 0

    lax.fori_loop(0, nround - 1, roundfn, 0)

    base = (nround - 1) * NBUF
    for u in range(NBUF):
        wait_g(base + u, u)
        fire_s(base + u, u)
    for u in range(NBUF):
        wait_s(base + u, u)


def _gather(x0, x1, tab_flat):
    mesh = plsc.VectorSubcoreMesh(core_axis_name="c", subcore_axis_name="s")
    scratch = [
        pltpu.VMEM((XB,), jnp.int32),
        pltpu.VMEM((XB,), jnp.int32),
        pltpu.VMEM((NCHUNK * CH,), jnp.int32),
        pltpu.VMEM_SHARED((NIDX * NIDX, EMBED), jnp.float32),
    ]
    scratch += [pltpu.VMEM((CH, EMBED), jnp.float32)] * NBUF
    scratch += [pltpu.SemaphoreType.DMA] * (2 * NBUF)
    f = pl.kernel(
        _gather_body,
        out_type=jax.ShapeDtypeStruct((N_TOKENS, EMBED), jnp.float32),
        mesh=mesh,
        scratch_types=scratch,
    )
    return f(x0, x1, tab_flat)


def kernel(x, token_table, pos_table):
    x2d = x.astype(jnp.int32).reshape(N_TOKENS, 2)
    x0 = x2d[:, 0]
    x1 = x2d[:, 1]
    tab = _build_table(token_table, pos_table[:NIDX])
    out = _gather(x0, x1, tab.reshape(NIDX * NIDX, EMBED))
    return out.reshape(16384, 100, EMBED)


# speedup vs baseline: 1.5465x; 1.0112x over previous
import math

import jax
import jax.numpy as jnp
from jax import lax
from jax.experimental import pallas as pl
from jax.experimental.pallas import tpu as pltpu
from jax.experimental.pallas import tpu_sc as plsc

EMBED = 128
NIDX = 23
SCALE = math.sqrt(EMBED)
NC, NS, L = 2, 16, 16
NW = NC * NS
N_TOKENS = 16384 * 100
TPW = N_TOKENS // NW
CH = 128
NCHUNK = TPW // CH
NBUF = 4
NROWS = N_TOKENS // 128


def _table_body(tok_ref, pos_ref, out_ref):
    tok = tok_ref[...]
    pos = pos_ref[...]
    out_ref[...] = (tok[:, None, :] + pos[None, :, :]) * SCALE


def _build_table(token_table, pos23):
    return pl.pallas_call(
        _table_body,
        out_shape=jax.ShapeDtypeStruct((NIDX, NIDX, EMBED), jnp.float32),
    )(token_table, pos23)


IDXBLK = 1600


def _idx_body(x0_ref, x1_ref, idx_ref):
    idx_ref[...] = x0_ref[...] * NIDX + x1_ref[...]


def _build_idx(x):
    x0 = x[..., 0].astype(jnp.int32).reshape(NROWS, 128)
    x1 = x[..., 1].astype(jnp.int32).reshape(NROWS, 128)
    idx = pl.pallas_call(
        _idx_body,
        grid=(NROWS // IDXBLK,),
        in_specs=[pl.BlockSpec((IDXBLK, 128), lambda i: (i, 0))] * 2,
        out_specs=pl.BlockSpec((IDXBLK, 128), lambda i: (i, 0)),
        out_shape=jax.ShapeDtypeStruct((NROWS, 128), jnp.int32),
    )(x0, x1)
    return idx.reshape(N_TOKENS)


def _gather_body(idx_hbm, tab_hbm, out_hbm, idxa, tab_sp, *rest):
    rows = list(rest[:NBUF])
    sgs = list(rest[NBUF:2 * NBUF])
    sss = list(rest[2 * NBUF:3 * NBUF])
    sid = lax.axis_index("s")
    wid = sid * NC + lax.axis_index("c")
    base_w = wid * TPW

    @pl.when(sid == 0)
    def _():
        pltpu.sync_copy(tab_hbm, tab_sp)

    pltpu.sync_copy(idx_hbm.at[pl.ds(base_w, TPW)], idxa)

    plsc.subcore_barrier()

    def idxs(i):
        return idxa.at[pl.ds(i * CH, CH)]

    def fire_g(i, b):
        pltpu.async_copy(tab_sp.at[idxs(i)], rows[b], sgs[b])

    def wait_g(i, b):
        pltpu.make_async_copy(tab_sp.at[idxs(i)], rows[b], sgs[b]).wait()

    def fire_s(i, b):
        pltpu.async_copy(rows[b], out_hbm.at[pl.ds(base_w + i * CH, CH)],
                         sss[b])

    def wait_s(i, b):
        pltpu.make_async_copy(rows[b],
                              out_hbm.at[pl.ds(base_w + i * CH, CH)],
                              sss[b]).wait()

    for u in range(NBUF):
        fire_g(u, u)

    nround = NCHUNK // NBUF

    def roundfn(r, _):
        base = r * NBUF
        for u in range(NBUF):
            wait_g(base + u, u)
            fire_s(base + u, u)
        for u in range(NBUF):
            wait_s(base + u, u)
            fire_g(base + u + NBUF, u)
        return 0

    lax.fori_loop(0, nround - 1, roundfn, 0)

    base = (nround - 1) * NBUF
    for u in range(NBUF):
        wait_g(base + u, u)
        fire_s(base + u, u)
    for u in range(NBUF):
        wait_s(base + u, u)


def _gather(idx, tab_flat):
    mesh = plsc.VectorSubcoreMesh(core_axis_name="c", subcore_axis_name="s")
    scratch = [
        pltpu.VMEM((TPW,), jnp.int32),
        pltpu.VMEM_SHARED((NIDX * NIDX, EMBED), jnp.float32),
    ]
    scratch += [pltpu.VMEM((CH, EMBED), jnp.float32)] * NBUF
    scratch += [pltpu.SemaphoreType.DMA] * (2 * NBUF)
    f = pl.kernel(
        _gather_body,
        out_type=jax.ShapeDtypeStruct((N_TOKENS, EMBED), jnp.float32),
        mesh=mesh,
        scratch_types=scratch,
    )
    return f(idx, tab_flat)


def kernel(x, token_table, pos_table):
    idx = _build_idx(x)
    tab = _build_table(token_table, pos_table[:NIDX])
    out = _gather(idx, tab.reshape(NIDX * NIDX, EMBED))
    return out.reshape(16384, 100, EMBED)
